# baseline (device time: 244848 ns/iter reference)
import jax
import jax.numpy as jnp
from jax import lax
from jax.experimental import pallas as pl
from jax.experimental.pallas import tpu as pltpu

CHUNK_ROWS = [256] * 15 + [16] * 16
K = len(CHUNK_ROWS)


def kernel(x):
    m, n = x.shape
    half = m // 2
    offs = [sum(CHUNK_ROWS[:k]) for k in range(K)]

    def body(x_ref, out_ref, stage, ld_sems, st_sems, send1, recv1, send2, recv2):
        my_x = lax.axis_index("x")
        my_y = lax.axis_index("y")
        y_peer = (my_x, 1 - my_y)
        x_peer = (1 - my_x, my_y)

        barrier = pltpu.get_barrier_semaphore()
        for nbr in (y_peer, x_peer):
            pl.semaphore_signal(
                barrier, inc=1, device_id=nbr, device_id_type=pl.DeviceIdType.MESH
            )
        pl.semaphore_wait(barrier, 2)

        h0 = my_x * half
        p1 = []
        for k in range(K):
            o, rck = offs[k], CHUNK_ROWS[k]
            rdma = pltpu.make_async_remote_copy(
                src_ref=x_ref.at[pl.ds(h0 + o, rck), :],
                dst_ref=out_ref.at[pl.ds(my_y * m + h0 + o, rck), :],
                send_sem=send1.at[k],
                recv_sem=recv1.at[k],
                device_id=y_peer,
                device_id_type=pl.DeviceIdType.MESH,
            )
            rdma.start()
            p1.append(rdma)

        lds = []
        sts = []
        for j in range(2):
            lds.append(
                pltpu.make_async_copy(
                    x_ref.at[pl.ds(j * half, half), :], stage.at[j], ld_sems.at[j]
                )
            )
            sts.append(
                pltpu.make_async_copy(
                    stage.at[j],
                    out_ref.at[pl.ds(my_y * m + j * half, half), :],
                    st_sems.at[j],
                )
            )
        lds[0].start()

        p2 = []
        for k in range(K):
            o, rck = offs[k], CHUNK_ROWS[k]
            p1[k].wait_recv()
            rdma = pltpu.make_async_remote_copy(
                src_ref=out_ref.at[pl.ds((1 - my_y) * m + h0 + o, rck), :],
                dst_ref=out_ref.at[pl.ds((1 - my_y) * m + h0 + o, rck), :],
                send_sem=send2.at[k],
                recv_sem=recv2.at[k],
                device_id=x_peer,
                device_id_type=pl.DeviceIdType.MESH,
            )
            rdma.start()
            p2.append(rdma)
            if k == 2:
                lds[0].wait()
                sts[0].start()
                lds[1].start()
            elif k == 6:
                lds[1].wait()
                sts[1].start()

        for k in range(K):
            p2[k].wait_recv()
        for k in range(K):
            p1[k].wait_send()
            p2[k].wait_send()
        sts[0].wait()
        sts[1].wait()

    return pl.pallas_call(
        body,
        out_shape=jax.ShapeDtypeStruct((2 * m, n), x.dtype),
        in_specs=[pl.BlockSpec(memory_space=pl.ANY)],
        out_specs=pl.BlockSpec(memory_space=pl.ANY),
        scratch_shapes=[
            pltpu.VMEM((2, half, n), x.dtype),
            pltpu.SemaphoreType.DMA((2,)),
            pltpu.SemaphoreType.DMA((2,)),
            pltpu.SemaphoreType.DMA((K,)),
            pltpu.SemaphoreType.DMA((K,)),
            pltpu.SemaphoreType.DMA((K,)),
            pltpu.SemaphoreType.DMA((K,)),
        ],
        compiler_params=pltpu.CompilerParams(collective_id=0),
    )(x)


# device time: 238956 ns/iter; 1.0247x vs baseline; 1.0247x over previous
import jax
import jax.numpy as jnp
from jax import lax
from jax.experimental import pallas as pl
from jax.experimental.pallas import tpu as pltpu

CHUNK_ROWS = [64] * 64
K = len(CHUNK_ROWS)


def kernel(x):
    m, n = x.shape
    half = m // 2
    offs = [sum(CHUNK_ROWS[:k]) for k in range(K)]

    def body(x_ref, out_ref, stage, ld_sems, st_sems, send1, recv1, send2, recv2):
        my_x = lax.axis_index("x")
        my_y = lax.axis_index("y")
        y_peer = (my_x, 1 - my_y)
        x_peer = (1 - my_x, my_y)

        barrier = pltpu.get_barrier_semaphore()
        for nbr in (y_peer, x_peer):
            pl.semaphore_signal(
                barrier, inc=1, device_id=nbr, device_id_type=pl.DeviceIdType.MESH
            )
        pl.semaphore_wait(barrier, 2)

        h0 = my_x * half
        p1 = []
        for k in range(K):
            o, rck = offs[k], CHUNK_ROWS[k]
            rdma = pltpu.make_async_remote_copy(
                src_ref=x_ref.at[pl.ds(h0 + o, rck), :],
                dst_ref=out_ref.at[pl.ds(my_y * m + h0 + o, rck), :],
                send_sem=send1.at[k],
                recv_sem=recv1.at[k],
                device_id=y_peer,
                device_id_type=pl.DeviceIdType.MESH,
            )
            rdma.start()
            p1.append(rdma)

        lds = []
        sts = []
        for j in range(2):
            lds.append(
                pltpu.make_async_copy(
                    x_ref.at[pl.ds(j * half, half), :], stage.at[j], ld_sems.at[j]
                )
            )
            sts.append(
                pltpu.make_async_copy(
                    stage.at[j],
                    out_ref.at[pl.ds(my_y * m + j * half, half), :],
                    st_sems.at[j],
                )
            )
        lds[0].start()

        p2 = []
        for k in range(K):
            o, rck = offs[k], CHUNK_ROWS[k]
            p1[k].wait_recv()
            rdma = pltpu.make_async_remote_copy(
                src_ref=out_ref.at[pl.ds((1 - my_y) * m + h0 + o, rck), :],
                dst_ref=out_ref.at[pl.ds((1 - my_y) * m + h0 + o, rck), :],
                send_sem=send2.at[k],
                recv_sem=recv2.at[k],
                device_id=x_peer,
                device_id_type=pl.DeviceIdType.MESH,
            )
            rdma.start()
            p2.append(rdma)
            if k == 2:
                lds[0].wait()
                sts[0].start()
                lds[1].start()
            elif k == 6:
                lds[1].wait()
                sts[1].start()

        for k in range(K):
            p2[k].wait_recv()
        for k in range(K):
            p1[k].wait_send()
            p2[k].wait_send()
        sts[0].wait()
        sts[1].wait()

    return pl.pallas_call(
        body,
        out_shape=jax.ShapeDtypeStruct((2 * m, n), x.dtype),
        in_specs=[pl.BlockSpec(memory_space=pl.ANY)],
        out_specs=pl.BlockSpec(memory_space=pl.ANY),
        scratch_shapes=[
            pltpu.VMEM((2, half, n), x.dtype),
            pltpu.SemaphoreType.DMA((2,)),
            pltpu.SemaphoreType.DMA((2,)),
            pltpu.SemaphoreType.DMA((K,)),
            pltpu.SemaphoreType.DMA((K,)),
            pltpu.SemaphoreType.DMA((K,)),
            pltpu.SemaphoreType.DMA((K,)),
        ],
        compiler_params=pltpu.CompilerParams(collective_id=0),
    )(x)
